# baseline (speedup 1.0000x reference)
import functools
import math

import jax
import jax.numpy as jnp
from jax import lax
from jax.experimental import pallas as pl
from jax.experimental.pallas import tpu as pltpu
from jax.experimental.pallas import tpu_sc as plsc

MIN_VAR = 0.02
MAX_VAR = 3.0

_ROWS_PER_CHUNK = 2


def _var_table_body(lv_ref, var_ref):
    var_ref[...] = jnp.minimum(jax.nn.softplus(lv_ref[...]) + MIN_VAR, MAX_VAR)


def _make_gather(n_rows, ids_per_row, pad_per_row, dim, nc, ns):
    nw = nc * ns
    rows_per_w = n_rows // nw
    n_chunks = rows_per_w // _ROWS_PER_CHUNK
    chunk_ids = _ROWS_PER_CHUNK * pad_per_row
    mesh = plsc.VectorSubcoreMesh(core_axis_name="c", subcore_axis_name="s")
    out_t = jax.ShapeDtypeStruct((n_rows, ids_per_row, dim), jnp.float32)

    @functools.partial(
        pl.kernel,
        out_type=(out_t, out_t),
        mesh=mesh,
        scratch_types=[
            pltpu.VMEM((rows_per_w * pad_per_row,), jnp.int32),
            pltpu.VMEM((2, chunk_ids, dim), jnp.float32),
            pltpu.VMEM((2, chunk_ids, dim), jnp.float32),
            pltpu.SemaphoreType.DMA,
            pltpu.SemaphoreType.DMA,
            pltpu.SemaphoreType.DMA,
            pltpu.SemaphoreType.DMA,
        ],
    )
    def gather_k(ids_hbm, mu_tab, var_tab, mu_out, var_out,
                 idx_all, mu_v, var_v, sg0, sg1, ss0, ss1):
        wid = lax.axis_index("s") * nc + lax.axis_index("c")
        row_base = wid * rows_per_w
        pltpu.sync_copy(ids_hbm.at[pl.ds(row_base * pad_per_row,
                                         rows_per_w * pad_per_row)], idx_all)
        sg = (sg0, sg1)
        ss = (ss0, ss1)

        def gather_pair(i, b):
            idx = idx_all.at[pl.ds(i * chunk_ids, chunk_ids)]
            return (pltpu.make_async_copy(mu_tab.at[idx], mu_v.at[b], sg[b]),
                    pltpu.make_async_copy(var_tab.at[idx], var_v.at[b], sg[b]))

        def scatter_pair(i, b):
            r0 = row_base + i * _ROWS_PER_CHUNK
            cps = []
            for j in range(_ROWS_PER_CHUNK):
                src = pl.ds(j * pad_per_row, ids_per_row)
                cps.append(pltpu.make_async_copy(
                    mu_v.at[b, src], mu_out.at[r0 + j], ss[b]))
                cps.append(pltpu.make_async_copy(
                    var_v.at[b, src], var_out.at[r0 + j], ss[b]))
            return cps

        def start(cps):
            for c in cps:
                c.start()

        def wait(cps):
            for c in cps:
                c.wait()

        start(gather_pair(0, 0))
        start(gather_pair(1, 1))
        wait(gather_pair(0, 0))
        start(scatter_pair(0, 0))

        def body(r, carry):
            for step in (1, 2):
                i = 2 * r + step
                b = step % 2
                wait(scatter_pair(i - 1, 1 - b))
                start(gather_pair(i + 1, 1 - b))
                wait(gather_pair(i, b))
                start(scatter_pair(i, b))
            return carry

        lax.fori_loop(0, (n_chunks - 2) // 2, body, 0)

        last = n_chunks - 1
        wait(gather_pair(last, last % 2))
        start(scatter_pair(last, last % 2))
        wait(scatter_pair(last - 1, (last - 1) % 2))
        wait(scatter_pair(last, last % 2))

    return gather_k


def kernel(ids, translation, log_var):
    var_table = pl.pallas_call(
        _var_table_body,
        out_shape=jax.ShapeDtypeStruct(log_var.shape, jnp.float32),
    )(log_var)

    info = plsc.get_sparse_core_info()
    n_rows, ids_per_row = ids.shape
    dim = translation.shape[1]
    pad_per_row = ((ids_per_row + 7) // 8) * 8
    ids_pad = jnp.pad(ids, ((0, 0), (0, pad_per_row - ids_per_row)))
    ids_flat = ids_pad.reshape(n_rows * pad_per_row)
    gather_k = _make_gather(n_rows, ids_per_row, pad_per_row, dim,
                            info.num_cores, info.num_subcores)
    return gather_k(ids_flat, translation, var_table)


# speedup vs baseline: 8.4070x; 8.4070x over previous
import functools
import math

import jax
import jax.numpy as jnp
from jax import lax
from jax.experimental import pallas as pl
from jax.experimental.pallas import tpu as pltpu
from jax.experimental.pallas import tpu_sc as plsc

MIN_VAR = 0.02
MAX_VAR = 3.0

_CHUNK = 128


def _var_table_body(lv_ref, var_ref):
    var_ref[...] = jnp.minimum(jax.nn.softplus(lv_ref[...]) + MIN_VAR, MAX_VAR)


def _make_gather(num_rows, dim, nc, ns):
    nw = nc * ns
    per_w = num_rows // nw
    n_chunks = per_w // _CHUNK
    mesh = plsc.VectorSubcoreMesh(core_axis_name="c", subcore_axis_name="s")
    out_t = jax.ShapeDtypeStruct((num_rows, dim), jnp.float32)

    @functools.partial(
        pl.kernel,
        out_type=(out_t, out_t),
        mesh=mesh,
        scratch_types=[
            pltpu.VMEM((per_w,), jnp.int32),
            pltpu.VMEM((2, _CHUNK, dim), jnp.float32),
            pltpu.VMEM((2, _CHUNK, dim), jnp.float32),
            pltpu.SemaphoreType.DMA,
            pltpu.SemaphoreType.DMA,
            pltpu.SemaphoreType.DMA,
            pltpu.SemaphoreType.DMA,
        ],
    )
    def gather_k(ids_hbm, mu_tab, var_tab, mu_out, var_out,
                 idx_all, mu_v, var_v, sg0, sg1, ss0, ss1):
        wid = lax.axis_index("s") * nc + lax.axis_index("c")
        base = wid * per_w
        pltpu.sync_copy(ids_hbm.at[pl.ds(base, per_w)], idx_all)
        sg = (sg0, sg1)
        ss = (ss0, ss1)

        def idx(i):
            return idx_all.at[pl.ds(i * _CHUNK, _CHUNK)]

        def gather_pair(i, b):
            return (pltpu.make_async_copy(mu_tab.at[idx(i)], mu_v.at[b], sg[b]),
                    pltpu.make_async_copy(var_tab.at[idx(i)], var_v.at[b], sg[b]))

        def scatter_pair(i, b):
            dst = pl.ds(base + i * _CHUNK, _CHUNK)
            return (pltpu.make_async_copy(mu_v.at[b], mu_out.at[dst], ss[b]),
                    pltpu.make_async_copy(var_v.at[b], var_out.at[dst], ss[b]))

        def start(pair):
            for c in pair:
                c.start()

        def wait(pair):
            for c in pair:
                c.wait()

        start(gather_pair(0, 0))
        start(gather_pair(1, 1))
        wait(gather_pair(0, 0))
        start(scatter_pair(0, 0))

        def body(r, carry):
            for step in (1, 2):
                i = 2 * r + step
                b = step % 2
                wait(scatter_pair(i - 1, 1 - b))
                start(gather_pair(i + 1, 1 - b))
                wait(gather_pair(i, b))
                start(scatter_pair(i, b))
            return carry

        lax.fori_loop(0, (n_chunks - 2) // 2, body, 0)

        last = n_chunks - 1
        wait(gather_pair(last, last % 2))
        start(scatter_pair(last, last % 2))
        wait(scatter_pair(last - 1, (last - 1) % 2))
        wait(scatter_pair(last, last % 2))

    return gather_k


def kernel(ids, translation, log_var):
    var_table = pl.pallas_call(
        _var_table_body,
        out_shape=jax.ShapeDtypeStruct(log_var.shape, jnp.float32),
    )(log_var)

    info = plsc.get_sparse_core_info()
    n_rows, ids_per_row = ids.shape
    num = ids.size
    dim = translation.shape[1]
    ids_flat = ids.T.reshape(num)
    gather_k = _make_gather(num, dim, info.num_cores, info.num_subcores)
    mu_flat, var_flat = gather_k(ids_flat, translation, var_table)
    mu = mu_flat.reshape(ids_per_row, n_rows, dim).transpose(1, 0, 2)
    var = var_flat.reshape(ids_per_row, n_rows, dim).transpose(1, 0, 2)
    return mu, var
